# baseline (speedup 1.0000x reference)
import functools

import jax
import jax.numpy as jnp
from jax import lax
from jax.experimental import pallas as pl
from jax.experimental.pallas import tpu as pltpu
from jax.experimental.pallas import tpu_sc as plsc

VOCAB = 1000
B, T = 1024, 200
N_TOK = B * T
NC, NS = 2, 16
NW = NC * NS
PER_W = N_TOK // NW
LANES = 16
KE = 128
NECHUNK = PER_W // KE

NVG = VOCAB // 8
NB = PER_W // 128
HALF = NB // 2
HTOK = HALF * 128


def _lse_body(table_ref, out_ref):
    x = table_ref[...]
    m = jnp.max(x, axis=1)
    s = jnp.sum(jnp.exp(x - m[:, None]), axis=1)
    out_ref[...] = m + jnp.log(s)


def _lse(table):
    return pl.pallas_call(
        _lse_body,
        out_shape=jax.ShapeDtypeStruct((VOCAB,), jnp.float32),
    )(table)


_MESH = plsc.VectorSubcoreMesh(core_axis_name="c", subcore_axis_name="s")


NTIL = N_TOK // 128
ROWW = NTIL * 1024
HWRD = HALF * 1024


@functools.partial(
    pl.kernel,
    out_type=jax.ShapeDtypeStruct((VOCAB * N_TOK,), jnp.float32),
    mesh=_MESH,
    compiler_params=pltpu.CompilerParams(
        needs_layout_passes=False, use_tc_tiling_on_sc=False),
    scratch_types=[
        pltpu.VMEM((PER_W,), jnp.int32),
        pltpu.VMEM((2 * 8 * VOCAB,), jnp.float32),
        pltpu.VMEM((2, HWRD), jnp.float32),
        pltpu.SemaphoreType.DMA((2,)),
        pltpu.SemaphoreType.DMA((2,)),
    ],
)
def _sc_gather_t(idx_hbm, ttr_hbm, out_hbm, idx_v, tbuf, obuf, gsem, ssem):
    wid = lax.axis_index("s") * NC + lax.axis_index("c")
    base = wid * PER_W
    pltpu.sync_copy(idx_hbm.at[pl.ds(base, PER_W)], idx_v)

    def fire_t(g, tb):
        pltpu.async_copy(ttr_hbm.at[pl.ds(g * 8 * VOCAB, 8 * VOCAB)],
                         tbuf.at[pl.ds(tb * 8 * VOCAB, 8 * VOCAB)],
                         gsem.at[tb])

    def wait_t(g, tb):
        pltpu.make_async_copy(
            ttr_hbm.at[pl.ds(g * 8 * VOCAB, 8 * VOCAB)],
            tbuf.at[pl.ds(tb * 8 * VOCAB, 8 * VOCAB)],
            gsem.at[tb]).wait()

    def _out_off(g, h):
        return g * ROWW + (wid * NB + h * HALF) * 1024

    def fire_sc(g, h):
        pltpu.async_copy(obuf.at[h],
                         out_hbm.at[pl.ds(_out_off(g, h), HWRD)],
                         ssem.at[h])

    def wait_sc(g, h):
        pltpu.make_async_copy(obuf.at[h],
                              out_hbm.at[pl.ds(_out_off(g, h), HWRD)],
                              ssem.at[h]).wait()

    def compute_half(h, tb):
        def bb_body(bb, _):
            for sub in range(8):
                o = (h * HALF + bb) * 128 + sub * 16
                iv = idx_v[pl.ds(o, LANES)]
                for v in range(8):
                    src = iv + (tb * 8 * VOCAB + v * VOCAB)
                    vals = plsc.load_gather(tbuf, [src])
                    obuf[h, pl.ds(bb * 1024 + v * 128 + sub * 16,
                                  LANES)] = vals
            return 0
        lax.fori_loop(0, HALF, bb_body, 0)

    fire_t(0, 0)
    fire_t(1, 1)
    wait_t(0, 0)
    for h in range(2):
        compute_half(h, 0)
        fire_sc(0, h)
    fire_t(2, 0)

    def g_body(g, _):
        tb = lax.rem(g, 2)
        wait_t(g, tb)
        for h in range(2):
            wait_sc(g - 1, h)
            compute_half(h, tb)
            fire_sc(g, h)
        fire_t(g + 2, tb)
        return 0

    lax.fori_loop(1, NVG - 2, g_body, 0)

    for g in (NVG - 2, NVG - 1):
        tb = g % 2
        wait_t(g, tb)
        for h in range(2):
            wait_sc(g - 1, h)
            compute_half(h, tb)
            fire_sc(g, h)

    for h in range(2):
        wait_sc(NVG - 1, h)


@functools.partial(
    pl.kernel,
    out_type=jax.ShapeDtypeStruct((NW * LANES,), jnp.float32),
    mesh=_MESH,
    compiler_params=pltpu.CompilerParams(
        needs_layout_passes=False, use_tc_tiling_on_sc=False),
    scratch_types=[
        pltpu.VMEM((PER_W,), jnp.int32),
        pltpu.VMEM((PER_W,), jnp.int32),
        pltpu.VMEM((2, KE), jnp.int32),
        pltpu.VMEM((2, KE), jnp.float32),
        pltpu.VMEM((VOCAB,), jnp.float32),
        pltpu.VMEM((LANES,), jnp.float32),
        pltpu.SemaphoreType.DMA((2,)),
    ],
)
def _sc_loss(idx_hbm, tgt_hbm, tflat_hbm, lse_hbm, part_hbm,
             idx_v, tgt_v, fid_v, val_v, lse_v, acc_v, vsem):
    wid = lax.axis_index("s") * NC + lax.axis_index("c")
    base = wid * PER_W
    pltpu.sync_copy(idx_hbm.at[pl.ds(base, PER_W)], idx_v)
    pltpu.sync_copy(tgt_hbm.at[pl.ds(base, PER_W)], tgt_v)
    pltpu.sync_copy(lse_hbm, lse_v)

    def build_fids(c, b):
        for g in range(KE // LANES):
            o = c * KE + g * LANES
            iv = idx_v[pl.ds(o, LANES)]
            cv = tgt_v[pl.ds(o, LANES)]
            fid_v[b, pl.ds(g * LANES, LANES)] = iv * VOCAB + cv

    def fire(c, b):
        pltpu.async_copy(
            tflat_hbm.at[fid_v.at[b]], val_v.at[b], vsem.at[b])

    def wait(c, b):
        pltpu.make_async_copy(
            tflat_hbm.at[fid_v.at[b]], val_v.at[b], vsem.at[b]).wait()

    def consume(c, b, acc):
        for g in range(KE // LANES):
            o = c * KE + g * LANES
            iv = idx_v[pl.ds(o, LANES)]
            lses = plsc.load_gather(lse_v, [iv])
            acc = acc + (lses - val_v[b, pl.ds(g * LANES, LANES)])
        return acc

    build_fids(0, 0)
    fire(0, 0)

    def step(c, acc):
        b = c % 2
        bn = 1 - b
        build_fids(c + 1, bn)
        fire(c + 1, bn)
        wait(c, b)
        return consume(c, b, acc)

    def outer(i, acc):
        c0 = i * 2
        acc = step(c0, acc)
        acc = step(c0 + 1, acc)
        return acc

    acc = jnp.zeros((LANES,), jnp.float32)
    n_steady = NECHUNK - 1
    n_outer = n_steady // 2
    acc = lax.fori_loop(0, n_outer, outer, acc)
    for c in range(n_outer * 2, NECHUNK - 1):
        acc = step(c, acc)
    c_last = NECHUNK - 1
    wait(c_last, c_last % 2)
    acc = consume(c_last, c_last % 2, acc)

    acc_v[...] = acc
    pltpu.sync_copy(acc_v, part_hbm.at[pl.ds(wid * LANES, LANES)])


def kernel(idx, targets, table):
    idx_f = idx.reshape(-1).astype(jnp.int32)
    tgt_f = targets.reshape(-1).astype(jnp.int32)
    ttr = table.T.reshape(-1)
    lse = _lse(table)
    out_flat = _sc_gather_t(idx_f, ttr)
    parts = _sc_loss(idx_f, tgt_f, table.reshape(-1), lse)
    loss = jnp.sum(parts) / N_TOK
    logits2 = (out_flat.reshape(NVG, NTIL, 8, 128)
               .transpose(1, 3, 0, 2).reshape(N_TOK, VOCAB))
    return (logits2, loss)


# speedup vs baseline: 2.4085x; 2.4085x over previous
import functools

import jax
import jax.numpy as jnp
from jax import lax
from jax.experimental import pallas as pl
from jax.experimental.pallas import tpu as pltpu
from jax.experimental.pallas import tpu_sc as plsc

VOCAB = 1000
B, T = 1024, 200
N_TOK = B * T
NC, NS = 2, 16
NW = NC * NS
PER_W = N_TOK // NW
LANES = 16
KE = 128
NECHUNK = PER_W // KE

NVG = VOCAB // 8
NB = PER_W // 128
HALF = NB // 2
HTOK = HALF * 128


def _lse_body(table_ref, out_ref):
    x = table_ref[...]
    m = jnp.max(x, axis=1)
    s = jnp.sum(jnp.exp(x - m[:, None]), axis=1)
    out_ref[...] = m + jnp.log(s)


def _lse(table):
    return pl.pallas_call(
        _lse_body,
        out_shape=jax.ShapeDtypeStruct((VOCAB,), jnp.float32),
    )(table)


_MESH = plsc.VectorSubcoreMesh(core_axis_name="c", subcore_axis_name="s")


NTIL = N_TOK // 128
ROWW = NTIL * 1024
HWRD = HALF * 1024


@functools.partial(
    pl.kernel,
    out_type=jax.ShapeDtypeStruct((VOCAB * N_TOK,), jnp.float32),
    mesh=_MESH,
    compiler_params=pltpu.CompilerParams(
        needs_layout_passes=False, use_tc_tiling_on_sc=False),
    scratch_types=[
        pltpu.VMEM((PER_W,), jnp.int32),
        pltpu.VMEM((2 * 8 * VOCAB,), jnp.float32),
        pltpu.VMEM((2, HWRD), jnp.float32),
        pltpu.SemaphoreType.DMA((2,)),
        pltpu.SemaphoreType.DMA((2,)),
    ],
)
def _sc_gather_t(idx_hbm, ttr_hbm, out_hbm, idx_v, tbuf, obuf, gsem, ssem):
    wid = lax.axis_index("s") * NC + lax.axis_index("c")
    base = wid * PER_W
    pltpu.sync_copy(idx_hbm.at[pl.ds(base, PER_W)], idx_v)

    def fire_t(g, tb):
        pltpu.async_copy(ttr_hbm.at[pl.ds(g * 8 * VOCAB, 8 * VOCAB)],
                         tbuf.at[pl.ds(tb * 8 * VOCAB, 8 * VOCAB)],
                         gsem.at[tb])

    def wait_t(g, tb):
        pltpu.make_async_copy(
            ttr_hbm.at[pl.ds(g * 8 * VOCAB, 8 * VOCAB)],
            tbuf.at[pl.ds(tb * 8 * VOCAB, 8 * VOCAB)],
            gsem.at[tb]).wait()

    def _out_off(g, h):
        return g * ROWW + (wid * NB + h * HALF) * 1024

    def fire_sc(g, h):
        pltpu.async_copy(obuf.at[h],
                         out_hbm.at[pl.ds(_out_off(g, h), HWRD)],
                         ssem.at[h])

    def wait_sc(g, h):
        pltpu.make_async_copy(obuf.at[h],
                              out_hbm.at[pl.ds(_out_off(g, h), HWRD)],
                              ssem.at[h]).wait()

    def compute_half(h, tb):
        tbbase = tb * (8 * VOCAB)

        def bb_body(bb, _):
            ob = bb * 1024
            o0 = (h * HALF + bb) * 128
            for sub in range(8):
                iv = idx_v[pl.ds(o0 + sub * 16, LANES)]
                src = iv + tbbase
                vals = [plsc.load_gather(tbuf, [src + v * VOCAB])
                        for v in range(8)]
                for v in range(8):
                    obuf[h, pl.ds(ob + v * 128 + sub * 16,
                                  LANES)] = vals[v]
            return 0
        lax.fori_loop(0, HALF, bb_body, 0)

    fire_t(0, 0)
    fire_t(1, 1)
    wait_t(0, 0)
    for h in range(2):
        compute_half(h, 0)
        fire_sc(0, h)
    fire_t(2, 0)

    def g_body(g, _):
        tb = lax.rem(g, 2)
        wait_t(g, tb)
        for h in range(2):
            wait_sc(g - 1, h)
            compute_half(h, tb)
            fire_sc(g, h)
        fire_t(g + 2, tb)
        return 0

    lax.fori_loop(1, NVG - 2, g_body, 0)

    for g in (NVG - 2, NVG - 1):
        tb = g % 2
        wait_t(g, tb)
        for h in range(2):
            wait_sc(g - 1, h)
            compute_half(h, tb)
            fire_sc(g, h)

    for h in range(2):
        wait_sc(NVG - 1, h)


@functools.partial(
    pl.kernel,
    out_type=jax.ShapeDtypeStruct((NW * LANES,), jnp.float32),
    mesh=_MESH,
    compiler_params=pltpu.CompilerParams(
        needs_layout_passes=False, use_tc_tiling_on_sc=False),
    scratch_types=[
        pltpu.VMEM((PER_W,), jnp.int32),
        pltpu.VMEM((PER_W,), jnp.int32),
        pltpu.VMEM((2, KE), jnp.int32),
        pltpu.VMEM((2, KE), jnp.float32),
        pltpu.VMEM((VOCAB,), jnp.float32),
        pltpu.VMEM((LANES,), jnp.float32),
        pltpu.SemaphoreType.DMA((2,)),
    ],
)
def _sc_loss(idx_hbm, tgt_hbm, tflat_hbm, lse_hbm, part_hbm,
             idx_v, tgt_v, fid_v, val_v, lse_v, acc_v, vsem):
    wid = lax.axis_index("s") * NC + lax.axis_index("c")
    base = wid * PER_W
    pltpu.sync_copy(idx_hbm.at[pl.ds(base, PER_W)], idx_v)
    pltpu.sync_copy(tgt_hbm.at[pl.ds(base, PER_W)], tgt_v)
    pltpu.sync_copy(lse_hbm, lse_v)

    def build_fids(c, b):
        for g in range(KE // LANES):
            o = c * KE + g * LANES
            iv = idx_v[pl.ds(o, LANES)]
            cv = tgt_v[pl.ds(o, LANES)]
            fid_v[b, pl.ds(g * LANES, LANES)] = iv * VOCAB + cv

    def fire(c, b):
        pltpu.async_copy(
            tflat_hbm.at[fid_v.at[b]], val_v.at[b], vsem.at[b])

    def wait(c, b):
        pltpu.make_async_copy(
            tflat_hbm.at[fid_v.at[b]], val_v.at[b], vsem.at[b]).wait()

    def consume(c, b, acc):
        for g in range(KE // LANES):
            o = c * KE + g * LANES
            iv = idx_v[pl.ds(o, LANES)]
            lses = plsc.load_gather(lse_v, [iv])
            acc = acc + (lses - val_v[b, pl.ds(g * LANES, LANES)])
        return acc

    build_fids(0, 0)
    fire(0, 0)

    def step(c, acc):
        b = c % 2
        bn = 1 - b
        build_fids(c + 1, bn)
        fire(c + 1, bn)
        wait(c, b)
        return consume(c, b, acc)

    def outer(i, acc):
        c0 = i * 2
        acc = step(c0, acc)
        acc = step(c0 + 1, acc)
        return acc

    acc = jnp.zeros((LANES,), jnp.float32)
    n_steady = NECHUNK - 1
    n_outer = n_steady // 2
    acc = lax.fori_loop(0, n_outer, outer, acc)
    for c in range(n_outer * 2, NECHUNK - 1):
        acc = step(c, acc)
    c_last = NECHUNK - 1
    wait(c_last, c_last % 2)
    acc = consume(c_last, c_last % 2, acc)

    acc_v[...] = acc
    pltpu.sync_copy(acc_v, part_hbm.at[pl.ds(wid * LANES, LANES)])


def kernel(idx, targets, table):
    idx_f = idx.reshape(-1).astype(jnp.int32)
    tgt_f = targets.reshape(-1).astype(jnp.int32)
    ttr = table.T.reshape(-1)
    lse = _lse(table)
    out_flat = _sc_gather_t(idx_f, ttr)
    parts = _sc_loss(idx_f, tgt_f, table.reshape(-1), lse)
    loss = jnp.sum(parts) / N_TOK
    logits2 = (out_flat.reshape(NVG, NTIL, 8, 128)
               .transpose(1, 3, 0, 2).reshape(N_TOK, VOCAB))
    return (logits2, loss)


# speedup vs baseline: 2.7688x; 1.1496x over previous
import functools

import jax
import jax.numpy as jnp
from jax import lax
from jax.experimental import pallas as pl
from jax.experimental.pallas import tpu as pltpu
from jax.experimental.pallas import tpu_sc as plsc

VOCAB = 1000
B, T = 1024, 200
N_TOK = B * T
NC, NS = 2, 16
NW = NC * NS
PER_W = N_TOK // NW
LANES = 16
KE = 128
NECHUNK = PER_W // KE

NVG = VOCAB // 8
NB = PER_W // 128
HALF = NB // 2
HTOK = HALF * 128


def _lse_body(table_ref, out_ref):
    x = table_ref[...]
    m = jnp.max(x, axis=1)
    s = jnp.sum(jnp.exp(x - m[:, None]), axis=1)
    out_ref[...] = m + jnp.log(s)


def _lse(table):
    return pl.pallas_call(
        _lse_body,
        out_shape=jax.ShapeDtypeStruct((VOCAB,), jnp.float32),
    )(table)


_MESH = plsc.VectorSubcoreMesh(core_axis_name="c", subcore_axis_name="s")


NTIL = N_TOK // 128
ROWW = NTIL * 1024
HWRD = HALF * 1024


@functools.partial(
    pl.kernel,
    out_type=jax.ShapeDtypeStruct((VOCAB * N_TOK,), jnp.float32),
    mesh=_MESH,
    compiler_params=pltpu.CompilerParams(
        needs_layout_passes=False, use_tc_tiling_on_sc=False),
    scratch_types=[
        pltpu.VMEM((PER_W,), jnp.int32),
        pltpu.VMEM((2, 8 * VOCAB), jnp.float32),
        pltpu.VMEM((2, HWRD), jnp.float32),
        pltpu.SemaphoreType.DMA((2,)),
        pltpu.SemaphoreType.DMA((2,)),
    ],
)
def _sc_gather_t(idx_hbm, ttr_hbm, out_hbm, idx_v, tbuf, obuf, gsem, ssem):
    wid = lax.axis_index("s") * NC + lax.axis_index("c")
    base = wid * PER_W
    pltpu.sync_copy(idx_hbm.at[pl.ds(base, PER_W)], idx_v)

    def fire_t(g, tb):
        pltpu.async_copy(ttr_hbm.at[pl.ds(g * 8 * VOCAB, 8 * VOCAB)],
                         tbuf.at[tb], gsem.at[tb])

    def wait_t(g, tb):
        pltpu.make_async_copy(
            ttr_hbm.at[pl.ds(g * 8 * VOCAB, 8 * VOCAB)],
            tbuf.at[tb], gsem.at[tb]).wait()

    def _out_off(g, h):
        return g * ROWW + (wid * NB + h * HALF) * 1024

    def fire_sc(g, h):
        pltpu.async_copy(obuf.at[h],
                         out_hbm.at[pl.ds(_out_off(g, h), HWRD)],
                         ssem.at[h])

    def wait_sc(g, h):
        pltpu.make_async_copy(obuf.at[h],
                              out_hbm.at[pl.ds(_out_off(g, h), HWRD)],
                              ssem.at[h]).wait()

    def compute_half(h, tb):
        def bb_body(bb, _):
            ob = bb * 1024
            o0 = (h * HALF + bb) * 128
            for sub in range(8):
                iv = idx_v[pl.ds(o0 + sub * 16, LANES)]
                vals = [plsc.load_gather(
                            tbuf.at[tb, pl.ds(v * VOCAB, VOCAB)], [iv])
                        for v in range(8)]
                for v in range(8):
                    obuf[h, pl.ds(ob + v * 128 + sub * 16,
                                  LANES)] = vals[v]
            return 0
        lax.fori_loop(0, HALF, bb_body, 0)

    fire_t(0, 0)
    fire_t(1, 1)
    wait_t(0, 0)
    for h in range(2):
        compute_half(h, 0)
        fire_sc(0, h)
    fire_t(2, 0)

    def g_body(g, _):
        tb = lax.rem(g, 2)
        wait_t(g, tb)
        for h in range(2):
            wait_sc(g - 1, h)
            compute_half(h, tb)
            fire_sc(g, h)
        fire_t(g + 2, tb)
        return 0

    lax.fori_loop(1, NVG - 2, g_body, 0)

    for g in (NVG - 2, NVG - 1):
        tb = g % 2
        wait_t(g, tb)
        for h in range(2):
            wait_sc(g - 1, h)
            compute_half(h, tb)
            fire_sc(g, h)

    for h in range(2):
        wait_sc(NVG - 1, h)


@functools.partial(
    pl.kernel,
    out_type=jax.ShapeDtypeStruct((NW * LANES,), jnp.float32),
    mesh=_MESH,
    compiler_params=pltpu.CompilerParams(
        needs_layout_passes=False, use_tc_tiling_on_sc=False),
    scratch_types=[
        pltpu.VMEM((PER_W,), jnp.int32),
        pltpu.VMEM((PER_W,), jnp.int32),
        pltpu.VMEM((2, KE), jnp.int32),
        pltpu.VMEM((2, KE), jnp.float32),
        pltpu.VMEM((VOCAB,), jnp.float32),
        pltpu.VMEM((LANES,), jnp.float32),
        pltpu.SemaphoreType.DMA((2,)),
    ],
)
def _sc_loss(idx_hbm, tgt_hbm, tflat_hbm, lse_hbm, part_hbm,
             idx_v, tgt_v, fid_v, val_v, lse_v, acc_v, vsem):
    wid = lax.axis_index("s") * NC + lax.axis_index("c")
    base = wid * PER_W
    pltpu.sync_copy(idx_hbm.at[pl.ds(base, PER_W)], idx_v)
    pltpu.sync_copy(tgt_hbm.at[pl.ds(base, PER_W)], tgt_v)
    pltpu.sync_copy(lse_hbm, lse_v)

    def build_fids(c, b):
        for g in range(KE // LANES):
            o = c * KE + g * LANES
            iv = idx_v[pl.ds(o, LANES)]
            cv = tgt_v[pl.ds(o, LANES)]
            fid_v[b, pl.ds(g * LANES, LANES)] = iv * VOCAB + cv

    def fire(c, b):
        pltpu.async_copy(
            tflat_hbm.at[fid_v.at[b]], val_v.at[b], vsem.at[b])

    def wait(c, b):
        pltpu.make_async_copy(
            tflat_hbm.at[fid_v.at[b]], val_v.at[b], vsem.at[b]).wait()

    def consume(c, b, acc):
        for g in range(KE // LANES):
            o = c * KE + g * LANES
            iv = idx_v[pl.ds(o, LANES)]
            lses = plsc.load_gather(lse_v, [iv])
            acc = acc + (lses - val_v[b, pl.ds(g * LANES, LANES)])
        return acc

    build_fids(0, 0)
    fire(0, 0)

    def step(c, acc):
        b = c % 2
        bn = 1 - b
        build_fids(c + 1, bn)
        fire(c + 1, bn)
        wait(c, b)
        return consume(c, b, acc)

    def outer(i, acc):
        c0 = i * 2
        acc = step(c0, acc)
        acc = step(c0 + 1, acc)
        return acc

    acc = jnp.zeros((LANES,), jnp.float32)
    n_steady = NECHUNK - 1
    n_outer = n_steady // 2
    acc = lax.fori_loop(0, n_outer, outer, acc)
    for c in range(n_outer * 2, NECHUNK - 1):
        acc = step(c, acc)
    c_last = NECHUNK - 1
    wait(c_last, c_last % 2)
    acc = consume(c_last, c_last % 2, acc)

    acc_v[...] = acc
    pltpu.sync_copy(acc_v, part_hbm.at[pl.ds(wid * LANES, LANES)])


def kernel(idx, targets, table):
    idx_f = idx.reshape(-1).astype(jnp.int32)
    tgt_f = targets.reshape(-1).astype(jnp.int32)
    ttr = table.T.reshape(-1)
    lse = _lse(table)
    out_flat = _sc_gather_t(idx_f, ttr)
    parts = _sc_loss(idx_f, tgt_f, table.reshape(-1), lse)
    loss = jnp.sum(parts) / N_TOK
    logits2 = (out_flat.reshape(NVG, NTIL, 8, 128)
               .transpose(1, 3, 0, 2).reshape(N_TOK, VOCAB))
    return (logits2, loss)
